# Initial kernel scaffold; baseline (speedup 1.0000x reference)
import jax
import jax.numpy as jnp
from jax.experimental import pallas as pl


def kernel(x, edge_index, W1, b1, W2, b2, W3, b3, Wl1, bl1, Wl2, bl2, Wl3, bl3):
    raise NotImplementedError("write your pallas kernel here")


# speedup vs baseline: 5.6383x; 5.6383x over previous
import functools

import jax
import jax.numpy as jnp
from jax import lax
from jax.experimental import pallas as pl
from jax.experimental.pallas import tpu as pltpu
from jax.experimental.pallas import tpu_sc as plsc

D = 128
CH = 128
NC = 2
NS = 16
NW = NC * NS
ACC_ROWS = 10240


def _sc_aggregate(table, src, dst, zeros_blk):
    n = table.shape[0]
    e = src.shape[0]
    nch = e // CH
    base_trips, rem = divmod(nch, NW)
    zrows = ACC_ROWS // NS

    mesh = plsc.VectorSubcoreMesh(
        core_axis_name="c", subcore_axis_name="s",
        num_cores=NC, num_subcores=NS)

    @functools.partial(
        pl.kernel,
        out_type=jax.ShapeDtypeStruct((NC, ACC_ROWS, D), jnp.float32),
        mesh=mesh,
        scratch_types=[
            pltpu.VMEM_SHARED((ACC_ROWS, D), jnp.float32),
            pltpu.VMEM((2, CH, D), jnp.float32),
            pltpu.VMEM((2, CH), jnp.int32),
            pltpu.VMEM((2, CH), jnp.int32),
            pltpu.SemaphoreType.DMA,
        ],
    )
    def agg(table_hbm, src_hbm, dst_hbm, zeros_hbm, out_hbm,
            acc, rows, sidx, didx, sem):
        cid = lax.axis_index("c")
        sid = lax.axis_index("s")
        wid = sid * NC + cid

        pltpu.sync_copy(zeros_hbm, rows.at[0])
        for k in range(zrows // CH):
            pltpu.sync_copy(rows.at[0], acc.at[pl.ds(sid * zrows + k * CH, CH)])
        plsc.subcore_barrier()

        ntr = base_trips + jnp.where(wid < rem, 1, 0).astype(jnp.int32)

        def body(i, carry):
            off = (wid + i * NW) * CH
            pltpu.sync_copy(src_hbm.at[pl.ds(off, CH)], sidx.at[0])
            pltpu.sync_copy(dst_hbm.at[pl.ds(off, CH)], didx.at[0])
            pltpu.async_copy(table_hbm.at[sidx.at[0]], rows.at[0], sem).wait()
            pltpu.sync_copy(rows.at[0], acc.at[didx.at[0]], add=True)
            return carry

        lax.fori_loop(0, ntr, body, 0)
        plsc.subcore_barrier()

        for k in range(zrows // CH):
            r0 = sid * zrows + k * CH
            pltpu.sync_copy(acc.at[pl.ds(r0, CH)], rows.at[0])
            pltpu.sync_copy(rows.at[0], out_hbm.at[cid, pl.ds(r0, CH)])

    return agg(table, src, dst, zeros_blk)[:, :n]


def _tc_layer(p0, p1, hprev, wt, wlt, bias, apply_elu):
    n = p0.shape[0]
    bn = 1000
    dout = wt.shape[1]

    def body(p0_r, p1_r, hp_r, wt_r, wlt_r, b_r, o_r):
        aggm = p0_r[...] + p1_r[...]
        y = jnp.dot(aggm, wt_r[...], preferred_element_type=jnp.float32)
        y = y + jnp.dot(hp_r[...], wlt_r[...], preferred_element_type=jnp.float32)
        y = y + b_r[...]
        if apply_elu:
            y = jnp.where(y > 0, y, jnp.exp(jnp.minimum(y, 0.0)) - 1.0)
        o_r[...] = y

    return pl.pallas_call(
        body,
        grid=(n // bn,),
        in_specs=[
            pl.BlockSpec((bn, D), lambda i: (i, 0)),
            pl.BlockSpec((bn, D), lambda i: (i, 0)),
            pl.BlockSpec((bn, D), lambda i: (i, 0)),
            pl.BlockSpec((D, dout), lambda i: (0, 0)),
            pl.BlockSpec((D, dout), lambda i: (0, 0)),
            pl.BlockSpec((1, dout), lambda i: (0, 0)),
        ],
        out_specs=pl.BlockSpec((bn, dout), lambda i: (i, 0)),
        out_shape=jax.ShapeDtypeStruct((n, dout), jnp.float32),
    )(p0, p1, hprev, wt, wlt, bias)


def kernel(x, edge_index, W1, b1, W2, b2, W3, b3,
           Wl1, bl1, Wl2, bl2, Wl3, bl3):
    src = edge_index[0]
    dst = edge_index[1]
    zeros_blk = jnp.zeros((CH, D), jnp.float32)

    agg1 = _sc_aggregate(x, src, dst, zeros_blk)
    h1 = _tc_layer(agg1[0], agg1[1], x, W1.T, Wl1.T, (b1 + bl1)[None, :], True)
    agg2 = _sc_aggregate(h1, src, dst, zeros_blk)
    h2 = _tc_layer(agg2[0], agg2[1], h1, W2.T, Wl2.T, (b2 + bl2)[None, :], True)
    agg3 = _sc_aggregate(h2, src, dst, zeros_blk)
    w3p = jnp.pad(W3, ((0, D - W3.shape[0]), (0, 0)))
    wl3p = jnp.pad(Wl3, ((0, D - Wl3.shape[0]), (0, 0)))
    b3p = jnp.pad(b3 + bl3, (0, D - b3.shape[0]))
    out = _tc_layer(agg3[0], agg3[1], h2, w3p.T, wl3p.T, b3p[None, :], False)
    return out[:, :W3.shape[0]]
